# Initial kernel scaffold; baseline (speedup 1.0000x reference)
import jax
import jax.numpy as jnp
from jax.experimental import pallas as pl


def kernel(x, edge_index, edge_attr, batch_mask, nn_W1, nn_b1, nn_W2, nn_b2, conv1_W, conv1_b, conv2_W, conv2_b, lin1_W, lin1_b, lin_mu_W, lin_mu_b):
    raise NotImplementedError("write your pallas kernel here")


# speedup vs baseline: 6.1428x; 6.1428x over previous
import functools

import jax
import jax.numpy as jnp
from jax import lax
from jax.experimental import pallas as pl
from jax.experimental.pallas import tpu as pltpu
from jax.experimental.pallas import tpu_sc as plsc

_N = 10000
_E = 320000
_D = 128
_G = 64
_NC = 2
_NS = 16
_NW = _NC * _NS
_EP = _E // _NW
_C = 80
_NCHUNK = _EP // _C
_RPT = _N // _NS
_ZR = 125
_NZ = _RPT // _ZR


def _sc_mesh():
    return plsc.VectorSubcoreMesh(core_axis_name="c", subcore_axis_name="s")


_SC_PARAMS = pltpu.CompilerParams(
    needs_layout_passes=False, use_tc_tiling_on_sc=False
)


@functools.partial(
    pl.kernel,
    out_type=jax.ShapeDtypeStruct((_NW, _N), jnp.float32),
    mesh=_sc_mesh(),
    compiler_params=_SC_PARAMS,
    scratch_types=[
        pltpu.VMEM((_N,), jnp.float32),
        pltpu.VMEM((_EP,), jnp.int32),
        pltpu.VMEM((_EP,), jnp.float32),
    ],
)
def _deg_kernel(col_hbm, ew_hbm, out_hbm, deg_v, col_v, ew_v):
    cid = lax.axis_index("c")
    sid = lax.axis_index("s")
    wid = sid * _NC + cid
    pltpu.sync_copy(col_hbm.at[wid], col_v)
    pltpu.sync_copy(ew_hbm.at[wid], ew_v)

    def zero_body(i, carry):
        deg_v[pl.ds(i * 16, 16)] = jnp.zeros((16,), jnp.float32)
        return carry

    lax.fori_loop(0, _N // 16, zero_body, 0)

    def acc_body(i, carry):
        c = col_v[pl.ds(i * 16, 16)]
        w = ew_v[pl.ds(i * 16, 16)]
        plsc.addupdate_scatter(deg_v, [c], w)
        return carry

    lax.fori_loop(0, _EP // 16, acc_body, 0)
    pltpu.sync_copy(deg_v, out_hbm.at[wid])


@functools.partial(
    pl.kernel,
    out_type=jax.ShapeDtypeStruct((_NC, _N, _D), jnp.float32),
    mesh=_sc_mesh(),
    compiler_params=_SC_PARAMS,
    scratch_types=[
        pltpu.VMEM((_C,), jnp.int32),
        pltpu.VMEM((_C,), jnp.int32),
        pltpu.VMEM((_C,), jnp.float32),
        pltpu.VMEM((_ZR, _D), jnp.float32),
        pltpu.VMEM_SHARED((_N, _D), jnp.float32),
        pltpu.SemaphoreType.DMA,
    ],
)
def _msg_kernel(row_hbm, col_hbm, ew_hbm, xs_hbm, out_hbm,
                row_v, col_v, ew_v, rows_v, acc_s, sem):
    cid = lax.axis_index("c")
    sid = lax.axis_index("s")
    wid = sid * _NC + cid

    def zb(i, carry):
        r = i // 8
        l = i % 8
        rows_v[r, pl.ds(l * 16, 16)] = jnp.zeros((16,), jnp.float32)
        return carry

    lax.fori_loop(0, _ZR * (_D // 16), zb, 0)

    def zs(i, carry):
        pltpu.sync_copy(rows_v, acc_s.at[pl.ds(sid * _RPT + i * _ZR, _ZR)])
        return carry

    lax.fori_loop(0, _NZ, zs, 0)
    plsc.subcore_barrier()

    def chunk(j, carry):
        base = j * _C
        pltpu.sync_copy(row_hbm.at[wid, pl.ds(base, _C)], row_v)
        pltpu.sync_copy(col_hbm.at[wid, pl.ds(base, _C)], col_v)
        pltpu.sync_copy(ew_hbm.at[wid, pl.ds(base, _C)], ew_v)
        dst = rows_v.at[pl.ds(0, _C)]
        pltpu.async_copy(xs_hbm.at[row_v], dst, sem).wait()

        def srow(k, c2):
            bidx = jnp.full((16,), k, jnp.int32)
            bw = plsc.load_gather(ew_v, [bidx])
            for m in range(_D // 16):
                sl = pl.ds(m * 16, 16)
                rows_v[k, sl] = rows_v[k, sl] * bw
            return c2

        lax.fori_loop(0, _C, srow, 0)
        pltpu.sync_copy(dst, acc_s.at[col_v], add=True)
        return carry

    lax.fori_loop(0, _NCHUNK, chunk, 0)
    plsc.subcore_barrier()

    def co(i, carry):
        base = sid * _RPT + i * _ZR
        pltpu.sync_copy(acc_s.at[pl.ds(base, _ZR)],
                        out_hbm.at[cid, pl.ds(base, _ZR)])
        return carry

    lax.fori_loop(0, _NZ, co, 0)


_EB = 3200


def _edge_mlp_body(ea_ref, w1_ref, b1_ref, w2_ref, b2_ref, out_ref):
    h = jnp.dot(ea_ref[...], w1_ref[...],
                preferred_element_type=jnp.float32,
                precision=lax.Precision.HIGHEST)
    h = jax.nn.relu(h + b1_ref[...])
    ew = jnp.dot(h, w2_ref[...],
                 preferred_element_type=jnp.float32,
                 precision=lax.Precision.HIGHEST) + b2_ref[...]
    out_ref[...] = ew


def _edge_mlp(edge_attr, w1, b1, w2, b2):
    grid = _E // _EB
    return pl.pallas_call(
        _edge_mlp_body,
        grid=(grid,),
        in_specs=[
            pl.BlockSpec((_EB, 16), lambda i: (i, 0)),
            pl.BlockSpec((16, 16), lambda i: (0, 0)),
            pl.BlockSpec((1, 16), lambda i: (0, 0)),
            pl.BlockSpec((16, 1), lambda i: (0, 0)),
            pl.BlockSpec((1, 1), lambda i: (0, 0)),
        ],
        out_specs=pl.BlockSpec((_EB, 1), lambda i: (i, 0)),
        out_shape=jax.ShapeDtypeStruct((_E, 1), jnp.float32),
    )(edge_attr, w1, b1.reshape(1, 16), w2, b2.reshape(1, 1))


def _prep_body(degp_ref, x_ref, w_ref, dinv_ref, xs_ref):
    deg = jnp.sum(degp_ref[...], axis=0) + 1.0
    dinv = jnp.where(deg > 0,
                     lax.rsqrt(jnp.maximum(deg, 1e-12)),
                     0.0)
    dinv_ref[...] = dinv.reshape(1, _N)
    xw = jnp.dot(x_ref[...], w_ref[...],
                 preferred_element_type=jnp.float32,
                 precision=lax.Precision.HIGHEST)
    xs_ref[...] = dinv.reshape(_N, 1) * xw


def _prep(deg_partials, x, conv1_w):
    return pl.pallas_call(
        _prep_body,
        out_shape=(
            jax.ShapeDtypeStruct((1, _N), jnp.float32),
            jax.ShapeDtypeStruct((_N, _D), jnp.float32),
        ),
    )(deg_partials, x, conv1_w)


def _mid_body(acc_ref, xs_ref, dinv_ref, b_ref, w_ref, xs2_ref):
    dinv = dinv_ref[...].reshape(_N, 1)
    tot = acc_ref[0] + acc_ref[1] + xs_ref[...]
    x1 = jax.nn.relu(dinv * tot + b_ref[...])
    xw2 = jnp.dot(x1, w_ref[...],
                  preferred_element_type=jnp.float32,
                  precision=lax.Precision.HIGHEST)
    xs2_ref[...] = dinv * xw2


def _mid(acc1, xs1, dinv, conv1_b, conv2_w):
    return pl.pallas_call(
        _mid_body,
        out_shape=jax.ShapeDtypeStruct((_N, _D), jnp.float32),
    )(acc1, xs1, dinv, conv1_b.reshape(1, _D), conv2_w)


def _tail_body(acc_ref, xs_ref, dinv_ref, b_ref, bm_ref,
               l1w_ref, l1b_ref, lmw_ref, lmb_ref, mu_ref):
    dinv = dinv_ref[...].reshape(_N, 1)
    tot = acc_ref[0] + acc_ref[1] + xs_ref[...]
    x2 = jax.nn.relu(dinv * tot + b_ref[...])
    seg = lax.broadcasted_iota(jnp.int32, (_G, _N), 0)
    onehot = (seg == bm_ref[...]).astype(jnp.float32)
    sums = jnp.dot(onehot, x2,
                   preferred_element_type=jnp.float32,
                   precision=lax.Precision.HIGHEST)
    cnt = jnp.sum(onehot, axis=1, keepdims=True)
    pooled = sums / jnp.maximum(cnt, 1.0)
    emb = jax.nn.relu(jnp.dot(pooled, l1w_ref[...],
                              preferred_element_type=jnp.float32,
                              precision=lax.Precision.HIGHEST) + l1b_ref[...])
    mu_ref[...] = jnp.dot(emb, lmw_ref[...],
                          preferred_element_type=jnp.float32,
                          precision=lax.Precision.HIGHEST) + lmb_ref[...]


def _tail(acc2, xs2, dinv, conv2_b, batch_mask, l1w, l1b, lmw, lmb):
    return pl.pallas_call(
        _tail_body,
        out_shape=jax.ShapeDtypeStruct((_G, 64), jnp.float32),
    )(acc2, xs2, dinv, conv2_b.reshape(1, _D), batch_mask.reshape(1, _N),
      l1w, l1b.reshape(1, _D), lmw, lmb.reshape(1, 64))


@jax.jit
def _run(x, edge_index, edge_attr, batch_mask, nn_W1, nn_b1, nn_W2, nn_b2,
         conv1_W, conv1_b, conv2_W, conv2_b, lin1_W, lin1_b,
         lin_mu_W, lin_mu_b):
    row = edge_index[0].reshape(_NW, _EP)
    col = edge_index[1].reshape(_NW, _EP)
    ew = _edge_mlp(edge_attr, nn_W1, nn_b1, nn_W2, nn_b2).reshape(_NW, _EP)
    deg_partials = _deg_kernel(col, ew)
    dinv, xs1 = _prep(deg_partials, x, conv1_W)
    acc1 = _msg_kernel(row, col, ew, xs1)
    xs2 = _mid(acc1, xs1, dinv, conv1_b, conv2_W)
    acc2 = _msg_kernel(row, col, ew, xs2)
    return _tail(acc2, xs2, dinv, conv2_b, batch_mask,
                 lin1_W, lin1_b, lin_mu_W, lin_mu_b)


def kernel(x, edge_index, edge_attr, batch_mask, nn_W1, nn_b1, nn_W2, nn_b2,
           conv1_W, conv1_b, conv2_W, conv2_b, lin1_W, lin1_b,
           lin_mu_W, lin_mu_b):
    return _run(x, edge_index, edge_attr, batch_mask, nn_W1, nn_b1, nn_W2,
                nn_b2, conv1_W, conv1_b, conv2_W, conv2_b, lin1_W, lin1_b,
                lin_mu_W, lin_mu_b)


# speedup vs baseline: 8.7892x; 1.4308x over previous
import functools

import jax
import jax.numpy as jnp
from jax import lax
from jax.experimental import pallas as pl
from jax.experimental.pallas import tpu as pltpu
from jax.experimental.pallas import tpu_sc as plsc

_N = 10000
_E = 320000
_D = 128
_G = 64
_NC = 2
_NS = 16
_NW = _NC * _NS
_EP = _E // _NW
_C = 80
_NCHUNK = _EP // _C
_RPT = _N // _NS
_ZR = 125
_NZ = _RPT // _ZR


def _sc_mesh():
    return plsc.VectorSubcoreMesh(core_axis_name="c", subcore_axis_name="s")


_SC_PARAMS = pltpu.CompilerParams(
    needs_layout_passes=False, use_tc_tiling_on_sc=False
)


@functools.partial(
    pl.kernel,
    out_type=jax.ShapeDtypeStruct((_NW, _N), jnp.float32),
    mesh=_sc_mesh(),
    compiler_params=_SC_PARAMS,
    scratch_types=[
        pltpu.VMEM((_N,), jnp.float32),
        pltpu.VMEM((_EP,), jnp.int32),
        pltpu.VMEM((_EP,), jnp.float32),
    ],
)
def _deg_kernel(col_hbm, ew_hbm, out_hbm, deg_v, col_v, ew_v):
    cid = lax.axis_index("c")
    sid = lax.axis_index("s")
    wid = sid * _NC + cid
    pltpu.sync_copy(col_hbm.at[wid], col_v)
    pltpu.sync_copy(ew_hbm.at[wid], ew_v)

    def zero_body(i, carry):
        deg_v[pl.ds(i * 16, 16)] = jnp.zeros((16,), jnp.float32)
        return carry

    lax.fori_loop(0, _N // 16, zero_body, 0)

    def acc_body(i, carry):
        c = col_v[pl.ds(i * 16, 16)]
        w = ew_v[pl.ds(i * 16, 16)]
        plsc.addupdate_scatter(deg_v, [c], w)
        return carry

    lax.fori_loop(0, _EP // 16, acc_body, 0)
    pltpu.sync_copy(deg_v, out_hbm.at[wid])


@functools.partial(
    pl.kernel,
    out_type=jax.ShapeDtypeStruct((_NC, _N, _D), jnp.float32),
    mesh=_sc_mesh(),
    compiler_params=_SC_PARAMS,
    scratch_types=[
        pltpu.VMEM((_C,), jnp.int32),
        pltpu.VMEM((_C,), jnp.int32),
        pltpu.VMEM((_C,), jnp.float32),
        pltpu.VMEM((_ZR, _D), jnp.float32),
        pltpu.VMEM_SHARED((_N, _D), jnp.float32),
        pltpu.SemaphoreType.DMA,
    ],
)
def _msg_kernel(row_hbm, col_hbm, ew_hbm, xs_hbm, out_hbm,
                row_v, col_v, ew_v, rows_v, acc_s, sem):
    cid = lax.axis_index("c")
    sid = lax.axis_index("s")
    wid = sid * _NC + cid

    def zb(i, carry):
        r = i // 8
        l = i % 8
        rows_v[r, pl.ds(l * 16, 16)] = jnp.zeros((16,), jnp.float32)
        return carry

    lax.fori_loop(0, _ZR * (_D // 16), zb, 0)

    def zs(i, carry):
        pltpu.sync_copy(rows_v, acc_s.at[pl.ds(sid * _RPT + i * _ZR, _ZR)])
        return carry

    lax.fori_loop(0, _NZ, zs, 0)
    plsc.subcore_barrier()

    def chunk(j, carry):
        base = j * _C
        pltpu.sync_copy(row_hbm.at[wid, pl.ds(base, _C)], row_v)
        pltpu.sync_copy(col_hbm.at[wid, pl.ds(base, _C)], col_v)
        pltpu.sync_copy(ew_hbm.at[wid, pl.ds(base, _C)], ew_v)
        dst = rows_v.at[pl.ds(0, _C)]
        pltpu.async_copy(xs_hbm.at[row_v], dst, sem).wait()

        def srow(k, c2):
            bidx = jnp.full((16,), k, jnp.int32)
            bw = plsc.load_gather(ew_v, [bidx])
            for m in range(_D // 16):
                sl = pl.ds(m * 16, 16)
                rows_v[k, sl] = rows_v[k, sl] * bw
            return c2

        lax.fori_loop(0, _C, srow, 0)
        pltpu.sync_copy(dst, acc_s.at[col_v], add=True)
        return carry

    lax.fori_loop(0, _NCHUNK, chunk, 0)
    plsc.subcore_barrier()

    def co(i, carry):
        base = sid * _RPT + i * _ZR
        pltpu.sync_copy(acc_s.at[pl.ds(base, _ZR)],
                        out_hbm.at[cid, pl.ds(base, _ZR)])
        return carry

    lax.fori_loop(0, _NZ, co, 0)


_EPACK = 8
_ER = _E // _EPACK
_EB = 5000


def _edge_mlp_body(ea_ref, w1_ref, b1_ref, w2_ref, b2_ref, out_ref):
    h = jnp.dot(ea_ref[...], w1_ref[...], preferred_element_type=jnp.float32)
    h = jax.nn.relu(h + b1_ref[...])
    ew = jnp.dot(h, w2_ref[...], preferred_element_type=jnp.float32) + b2_ref[...]
    out_ref[...] = ew


def _edge_mlp(edge_attr, w1, b1, w2, b2):
    ea2 = edge_attr.reshape(_ER, _EPACK * 16)
    eye = jnp.eye(_EPACK, dtype=jnp.float32)
    w1b = jnp.einsum("pq,io->piqo", eye, w1).reshape(_EPACK * 16, _EPACK * 16)
    b1t = jnp.tile(b1, _EPACK).reshape(1, _EPACK * 16)
    w2b = jnp.einsum("pq,io->piqo", eye, w2).reshape(_EPACK * 16, _EPACK)
    b2t = jnp.tile(b2, _EPACK).reshape(1, _EPACK)
    grid = _ER // _EB
    return pl.pallas_call(
        _edge_mlp_body,
        grid=(grid,),
        in_specs=[
            pl.BlockSpec((_EB, _EPACK * 16), lambda i: (i, 0)),
            pl.BlockSpec((_EPACK * 16, _EPACK * 16), lambda i: (0, 0)),
            pl.BlockSpec((1, _EPACK * 16), lambda i: (0, 0)),
            pl.BlockSpec((_EPACK * 16, _EPACK), lambda i: (0, 0)),
            pl.BlockSpec((1, _EPACK), lambda i: (0, 0)),
        ],
        out_specs=pl.BlockSpec((_EB, _EPACK), lambda i: (i, 0)),
        out_shape=jax.ShapeDtypeStruct((_ER, _EPACK), jnp.float32),
    )(ea2, w1b, b1t, w2b, b2t)


def _prep_body(degp_ref, x_ref, w_ref, dinv_ref, xs_ref):
    deg = jnp.sum(degp_ref[...], axis=0) + 1.0
    dinv = jnp.where(deg > 0,
                     lax.rsqrt(jnp.maximum(deg, 1e-12)),
                     0.0)
    dinv_ref[...] = dinv.reshape(1, _N)
    xw = jnp.dot(x_ref[...], w_ref[...], preferred_element_type=jnp.float32)
    xs_ref[...] = dinv.reshape(_N, 1) * xw


def _prep(deg_partials, x, conv1_w):
    return pl.pallas_call(
        _prep_body,
        out_shape=(
            jax.ShapeDtypeStruct((1, _N), jnp.float32),
            jax.ShapeDtypeStruct((_N, _D), jnp.float32),
        ),
    )(deg_partials, x, conv1_w)


def _mid_body(acc_ref, xs_ref, dinv_ref, b_ref, w_ref, xs2_ref):
    dinv = dinv_ref[...].reshape(_N, 1)
    tot = acc_ref[0] + acc_ref[1] + xs_ref[...]
    x1 = jax.nn.relu(dinv * tot + b_ref[...])
    xw2 = jnp.dot(x1, w_ref[...], preferred_element_type=jnp.float32)
    xs2_ref[...] = dinv * xw2


def _mid(acc1, xs1, dinv, conv1_b, conv2_w):
    return pl.pallas_call(
        _mid_body,
        out_shape=jax.ShapeDtypeStruct((_N, _D), jnp.float32),
    )(acc1, xs1, dinv, conv1_b.reshape(1, _D), conv2_w)


def _tail_body(acc_ref, xs_ref, dinv_ref, b_ref, bm_ref,
               l1w_ref, l1b_ref, lmw_ref, lmb_ref, mu_ref):
    dinv = dinv_ref[...].reshape(_N, 1)
    tot = acc_ref[0] + acc_ref[1] + xs_ref[...]
    x2 = jax.nn.relu(dinv * tot + b_ref[...])
    seg = lax.broadcasted_iota(jnp.int32, (_G, _N), 0)
    onehot = (seg == bm_ref[...]).astype(jnp.float32)
    sums = jnp.dot(onehot, x2, preferred_element_type=jnp.float32)
    cnt = jnp.sum(onehot, axis=1, keepdims=True)
    pooled = sums / jnp.maximum(cnt, 1.0)
    emb = jax.nn.relu(jnp.dot(pooled, l1w_ref[...], preferred_element_type=jnp.float32) + l1b_ref[...])
    mu_ref[...] = jnp.dot(emb, lmw_ref[...], preferred_element_type=jnp.float32) + lmb_ref[...]


def _tail(acc2, xs2, dinv, conv2_b, batch_mask, l1w, l1b, lmw, lmb):
    return pl.pallas_call(
        _tail_body,
        out_shape=jax.ShapeDtypeStruct((_G, 64), jnp.float32),
    )(acc2, xs2, dinv, conv2_b.reshape(1, _D), batch_mask.reshape(1, _N),
      l1w, l1b.reshape(1, _D), lmw, lmb.reshape(1, 64))


@jax.jit
def _run(x, edge_index, edge_attr, batch_mask, nn_W1, nn_b1, nn_W2, nn_b2,
         conv1_W, conv1_b, conv2_W, conv2_b, lin1_W, lin1_b,
         lin_mu_W, lin_mu_b):
    row = edge_index[0].reshape(_NW, _EP)
    col = edge_index[1].reshape(_NW, _EP)
    ew = _edge_mlp(edge_attr, nn_W1, nn_b1, nn_W2, nn_b2).reshape(_NW, _EP)
    deg_partials = _deg_kernel(col, ew)
    dinv, xs1 = _prep(deg_partials, x, conv1_W)
    acc1 = _msg_kernel(row, col, ew, xs1)
    xs2 = _mid(acc1, xs1, dinv, conv1_b, conv2_W)
    acc2 = _msg_kernel(row, col, ew, xs2)
    return _tail(acc2, xs2, dinv, conv2_b, batch_mask,
                 lin1_W, lin1_b, lin_mu_W, lin_mu_b)


def kernel(x, edge_index, edge_attr, batch_mask, nn_W1, nn_b1, nn_W2, nn_b2,
           conv1_W, conv1_b, conv2_W, conv2_b, lin1_W, lin1_b,
           lin_mu_W, lin_mu_b):
    return _run(x, edge_index, edge_attr, batch_mask, nn_W1, nn_b1, nn_W2,
                nn_b2, conv1_W, conv1_b, conv2_W, conv2_b, lin1_W, lin1_b,
                lin_mu_W, lin_mu_b)


# speedup vs baseline: 14.0683x; 1.6006x over previous
import functools

import jax
import jax.numpy as jnp
from jax import lax
from jax.experimental import pallas as pl
from jax.experimental.pallas import tpu as pltpu
from jax.experimental.pallas import tpu_sc as plsc

_N = 10000
_E = 320000
_D = 128
_G = 64
_NC = 2
_NS = 16
_NW = _NC * _NS
_EP = _E // _NW
_C = 80
_NCHUNK = _EP // _C
_RPT = _N // _NS
_ZR = 125
_NZ = _RPT // _ZR


def _sc_mesh():
    return plsc.VectorSubcoreMesh(core_axis_name="c", subcore_axis_name="s")


_SC_PARAMS = pltpu.CompilerParams(
    needs_layout_passes=False, use_tc_tiling_on_sc=False
)


@functools.partial(
    pl.kernel,
    out_type=jax.ShapeDtypeStruct((_NW, _N), jnp.float32),
    mesh=_sc_mesh(),
    compiler_params=_SC_PARAMS,
    scratch_types=[
        pltpu.VMEM((_N,), jnp.float32),
        pltpu.VMEM((_EP,), jnp.int32),
        pltpu.VMEM((_EP,), jnp.float32),
    ],
)
def _deg_kernel(col_hbm, ew_hbm, out_hbm, deg_v, col_v, ew_v):
    cid = lax.axis_index("c")
    sid = lax.axis_index("s")
    wid = sid * _NC + cid
    pltpu.sync_copy(col_hbm.at[wid], col_v)
    pltpu.sync_copy(ew_hbm.at[wid], ew_v)

    def zero_body(i, carry):
        deg_v[pl.ds(i * 16, 16)] = jnp.zeros((16,), jnp.float32)
        return carry

    lax.fori_loop(0, _N // 16, zero_body, 0)

    def acc_body(i, carry):
        c = col_v[pl.ds(i * 16, 16)]
        w = ew_v[pl.ds(i * 16, 16)]
        plsc.addupdate_scatter(deg_v, [c], w)
        return carry

    lax.fori_loop(0, _EP // 16, acc_body, 0)
    pltpu.sync_copy(deg_v, out_hbm.at[wid])


@functools.partial(
    pl.kernel,
    out_type=jax.ShapeDtypeStruct((_NC, _N, _D), jnp.float32),
    mesh=_sc_mesh(),
    compiler_params=_SC_PARAMS,
    scratch_types=[
        pltpu.VMEM((2, _C), jnp.int32),
        pltpu.VMEM((2, _C), jnp.int32),
        pltpu.VMEM((2, _C), jnp.int32),
        pltpu.VMEM((_C, _D), jnp.float32),
        pltpu.VMEM((_C, _D), jnp.float32),
        pltpu.VMEM((_C, _D), jnp.float32),
        pltpu.VMEM((_C, 16), jnp.float32),
        pltpu.VMEM((_C, 16), jnp.float32),
        pltpu.VMEM((_C, 16), jnp.float32),
        pltpu.VMEM_SHARED((_N, _D), jnp.float32),
        pltpu.SemaphoreType.DMA, pltpu.SemaphoreType.DMA,
        pltpu.SemaphoreType.DMA, pltpu.SemaphoreType.DMA,
        pltpu.SemaphoreType.DMA, pltpu.SemaphoreType.DMA,
        pltpu.SemaphoreType.DMA, pltpu.SemaphoreType.DMA,
        pltpu.SemaphoreType.DMA, pltpu.SemaphoreType.DMA,
        pltpu.SemaphoreType.DMA, pltpu.SemaphoreType.DMA,
    ],
)
def _msg_kernel(rc_hbm, ewb_hbm, xs_hbm, out_hbm,
                ib0, ib1, ib2, gb0, gb1, gb2, eb0, eb1, eb2, acc_s,
                is0, is1, is2, gs0, gs1, gs2, es0, es1, es2,
                ss0, ss1, ss2):
    cid = lax.axis_index("c")
    sid = lax.axis_index("s")
    wid = sid * _NC + cid
    ib = (ib0, ib1, ib2)
    gb = (gb0, gb1, gb2)
    eb = (eb0, eb1, eb2)
    isem = (is0, is1, is2)
    gsem = (gs0, gs1, gs2)
    esem = (es0, es1, es2)
    ssem = (ss0, ss1, ss2)

    def zb(i, carry):
        r = i // 8
        l = i % 8
        gb0[r, pl.ds(l * 16, 16)] = jnp.zeros((16,), jnp.float32)
        return carry

    lax.fori_loop(0, _C * (_D // 16), zb, 0)

    def zs(i, carry):
        pltpu.sync_copy(gb0, acc_s.at[pl.ds(sid * _RPT + i * _C, _C)])
        return carry

    lax.fori_loop(0, _RPT // _C, zs, 0)
    rem = _RPT % _C
    if rem:
        pltpu.sync_copy(gb0.at[pl.ds(0, rem)],
                        acc_s.at[pl.ds(sid * _RPT + (_RPT // _C) * _C, rem)])
    plsc.subcore_barrier()

    def start_idx(j, b):
        pltpu.async_copy(rc_hbm.at[wid, j], ib[b], isem[b])

    def start_gather(j, b):
        pltpu.async_copy(xs_hbm.at[ib[b].at[0]], gb[b], gsem[b])
        pltpu.async_copy(ewb_hbm.at[wid, j], eb[b], esem[b])

    def wait_gather(b):
        pltpu.make_async_copy(xs_hbm.at[ib[b].at[0]], gb[b], gsem[b]).wait()
        pltpu.make_async_copy(ewb_hbm.at[wid, 0], eb[b], esem[b]).wait()

    def scale(b):
        def srow(k, carry):
            bw = eb[b][k, pl.ds(0, 16)]
            for m in range(_D // 16):
                sl = pl.ds(m * 16, 16)
                gb[b][k, sl] = gb[b][k, sl] * bw
            return carry

        lax.fori_loop(0, _C, srow, 0)

    def start_scatter(j, b):
        pltpu.async_copy(gb[b], acc_s.at[ib[b].at[1]], ssem[b], add=True)

    def wait_scatter(b):
        pltpu.make_async_copy(gb[b], acc_s.at[ib[b].at[1]], ssem[b]).wait()

    start_idx(0, 0)
    pltpu.make_async_copy(rc_hbm.at[wid, 0], ib[0], isem[0]).wait()
    start_idx(1, 1)
    start_gather(0, 0)

    def step(j, x, p, n, first, tail):
        if not tail:
            @pl.when(j < _NCHUNK - 1)
            def _():
                pltpu.make_async_copy(rc_hbm.at[wid, 0], ib[n], isem[n]).wait()
                start_gather(j + 1, n)
        wait_gather(x)
        scale(x)
        if not first:
            wait_scatter(p)
        if not tail:
            @pl.when(j < _NCHUNK - 2)
            def _():
                start_idx(j + 2, p)
        start_scatter(j, x)

    def triple(j3, carry):
        j = j3 * 3
        step(j, 0, 2, 1, False, False)
        step(j + 1, 1, 0, 2, False, False)
        step(j + 2, 2, 1, 0, False, False)
        return carry

    step(0, 0, 2, 1, True, False)
    step(1, 1, 0, 2, False, False)
    step(2, 2, 1, 0, False, False)
    lax.fori_loop(1, _NCHUNK // 3, triple, 0)
    step(_NCHUNK - 2, (_NCHUNK - 2) % 3, (_NCHUNK - 3) % 3,
         (_NCHUNK - 1) % 3, False, False)
    step(_NCHUNK - 1, (_NCHUNK - 1) % 3, (_NCHUNK - 2) % 3,
         _NCHUNK % 3, False, True)
    wait_scatter((_NCHUNK - 1) % 3)
    plsc.subcore_barrier()

    def co(i, carry):
        base = sid * _RPT + i * _ZR
        pltpu.sync_copy(acc_s.at[pl.ds(base, _ZR)],
                        out_hbm.at[cid, pl.ds(base, _ZR)])
        return carry

    lax.fori_loop(0, _NZ, co, 0)


_EPACK = 8
_ER = _E // _EPACK
_EB = 5000


def _edge_mlp_body(ea_ref, w1_ref, b1_ref, w2_ref, b2_ref, out_ref):
    h = jnp.dot(ea_ref[...], w1_ref[...], preferred_element_type=jnp.float32)
    h = jax.nn.relu(h + b1_ref[...])
    ew = jnp.dot(h, w2_ref[...], preferred_element_type=jnp.float32) + b2_ref[...]
    out_ref[...] = ew


def _edge_mlp(edge_attr, w1, b1, w2, b2):
    ea2 = edge_attr.reshape(_ER, _EPACK * 16)
    eye = jnp.eye(_EPACK, dtype=jnp.float32)
    w1b = jnp.einsum("pq,io->piqo", eye, w1).reshape(_EPACK * 16, _EPACK * 16)
    b1t = jnp.tile(b1, _EPACK).reshape(1, _EPACK * 16)
    ones16 = jnp.ones((16,), jnp.float32)
    w2b = jnp.einsum("pq,i,m->piqm", eye, w2[:, 0],
                     ones16).reshape(_EPACK * 16, _EPACK * 16)
    b2t = jnp.broadcast_to(b2, (1, _EPACK * 16))
    grid = _ER // _EB
    return pl.pallas_call(
        _edge_mlp_body,
        grid=(grid,),
        in_specs=[
            pl.BlockSpec((_EB, _EPACK * 16), lambda i: (i, 0)),
            pl.BlockSpec((_EPACK * 16, _EPACK * 16), lambda i: (0, 0)),
            pl.BlockSpec((1, _EPACK * 16), lambda i: (0, 0)),
            pl.BlockSpec((_EPACK * 16, _EPACK * 16), lambda i: (0, 0)),
            pl.BlockSpec((1, _EPACK * 16), lambda i: (0, 0)),
        ],
        out_specs=pl.BlockSpec((_EB, _EPACK * 16), lambda i: (i, 0)),
        out_shape=jax.ShapeDtypeStruct((_ER, _EPACK * 16), jnp.float32),
    )(ea2, w1b, b1t, w2b, b2t)


def _prep_body(degp_ref, x_ref, w_ref, dinv_ref, xs_ref):
    deg = jnp.sum(degp_ref[...], axis=0) + 1.0
    dinv = jnp.where(deg > 0,
                     lax.rsqrt(jnp.maximum(deg, 1e-12)),
                     0.0)
    dinv_ref[...] = dinv.reshape(1, _N)
    xw = jnp.dot(x_ref[...], w_ref[...], preferred_element_type=jnp.float32)
    xs_ref[...] = dinv.reshape(_N, 1) * xw


def _prep(deg_partials, x, conv1_w):
    return pl.pallas_call(
        _prep_body,
        out_shape=(
            jax.ShapeDtypeStruct((1, _N), jnp.float32),
            jax.ShapeDtypeStruct((_N, _D), jnp.float32),
        ),
    )(deg_partials, x, conv1_w)


def _mid_body(acc_ref, xs_ref, dinv_ref, b_ref, w_ref, xs2_ref):
    dinv = dinv_ref[...].reshape(_N, 1)
    tot = acc_ref[0] + acc_ref[1] + xs_ref[...]
    x1 = jax.nn.relu(dinv * tot + b_ref[...])
    xw2 = jnp.dot(x1, w_ref[...], preferred_element_type=jnp.float32)
    xs2_ref[...] = dinv * xw2


def _mid(acc1, xs1, dinv, conv1_b, conv2_w):
    return pl.pallas_call(
        _mid_body,
        out_shape=jax.ShapeDtypeStruct((_N, _D), jnp.float32),
    )(acc1, xs1, dinv, conv1_b.reshape(1, _D), conv2_w)


def _tail_body(acc_ref, xs_ref, dinv_ref, b_ref, bm_ref,
               l1w_ref, l1b_ref, lmw_ref, lmb_ref, mu_ref):
    dinv = dinv_ref[...].reshape(_N, 1)
    tot = acc_ref[0] + acc_ref[1] + xs_ref[...]
    x2 = jax.nn.relu(dinv * tot + b_ref[...])
    seg = lax.broadcasted_iota(jnp.int32, (_G, _N), 0)
    onehot = (seg == bm_ref[...]).astype(jnp.float32)
    sums = jnp.dot(onehot, x2, preferred_element_type=jnp.float32)
    cnt = jnp.sum(onehot, axis=1, keepdims=True)
    pooled = sums / jnp.maximum(cnt, 1.0)
    emb = jax.nn.relu(jnp.dot(pooled, l1w_ref[...], preferred_element_type=jnp.float32) + l1b_ref[...])
    mu_ref[...] = jnp.dot(emb, lmw_ref[...], preferred_element_type=jnp.float32) + lmb_ref[...]


def _tail(acc2, xs2, dinv, conv2_b, batch_mask, l1w, l1b, lmw, lmb):
    return pl.pallas_call(
        _tail_body,
        out_shape=jax.ShapeDtypeStruct((_G, 64), jnp.float32),
    )(acc2, xs2, dinv, conv2_b.reshape(1, _D), batch_mask.reshape(1, _N),
      l1w, l1b.reshape(1, _D), lmw, lmb.reshape(1, 64))


@jax.jit
def _run(x, edge_index, edge_attr, batch_mask, nn_W1, nn_b1, nn_W2, nn_b2,
         conv1_W, conv1_b, conv2_W, conv2_b, lin1_W, lin1_b,
         lin_mu_W, lin_mu_b):
    row = edge_index[0]
    col = edge_index[1]
    rc = jnp.stack([row.reshape(_NW, _NCHUNK, _C),
                    col.reshape(_NW, _NCHUNK, _C)], axis=2)
    ew16 = _edge_mlp(edge_attr, nn_W1, nn_b1, nn_W2, nn_b2).reshape(_E, 16)
    ewb4 = ew16.reshape(_NW, _NCHUNK, _C, 16)
    ew = ew16[:, 0].reshape(_NW, _EP)
    deg_partials = _deg_kernel(col.reshape(_NW, _EP), ew)
    dinv, xs1 = _prep(deg_partials, x, conv1_W)
    acc1 = _msg_kernel(rc, ewb4, xs1)
    xs2 = _mid(acc1, xs1, dinv, conv1_b, conv2_W)
    acc2 = _msg_kernel(rc, ewb4, xs2)
    return _tail(acc2, xs2, dinv, conv2_b, batch_mask,
                 lin1_W, lin1_b, lin_mu_W, lin_mu_b)


def kernel(x, edge_index, edge_attr, batch_mask, nn_W1, nn_b1, nn_W2, nn_b2,
           conv1_W, conv1_b, conv2_W, conv2_b, lin1_W, lin1_b,
           lin_mu_W, lin_mu_b):
    return _run(x, edge_index, edge_attr, batch_mask, nn_W1, nn_b1, nn_W2,
                nn_b2, conv1_W, conv1_b, conv2_W, conv2_b, lin1_W, lin1_b,
                lin_mu_W, lin_mu_b)


# speedup vs baseline: 17.9890x; 1.2787x over previous
import functools

import jax
import jax.numpy as jnp
from jax import lax
from jax.experimental import pallas as pl
from jax.experimental.pallas import tpu as pltpu
from jax.experimental.pallas import tpu_sc as plsc

_N = 10000
_E = 320000
_D = 128
_G = 64
_NC = 2
_NS = 16
_NW = _NC * _NS
_EP = _E // _NW
_C = 80
_NCHUNK = _EP // _C
_RPT = _N // _NS
_ZR = 125
_NZ = _RPT // _ZR


def _sc_mesh():
    return plsc.VectorSubcoreMesh(core_axis_name="c", subcore_axis_name="s")


_SC_PARAMS = pltpu.CompilerParams(
    needs_layout_passes=False, use_tc_tiling_on_sc=False
)


@functools.partial(
    pl.kernel,
    out_type=jax.ShapeDtypeStruct((_NW, _N), jnp.float32),
    mesh=_sc_mesh(),
    compiler_params=_SC_PARAMS,
    scratch_types=[
        pltpu.VMEM((_N,), jnp.float32),
        pltpu.VMEM((_EP,), jnp.int32),
        pltpu.VMEM((_EP,), jnp.float32),
    ],
)
def _deg_kernel(col_hbm, ew_hbm, out_hbm, deg_v, col_v, ew_v):
    cid = lax.axis_index("c")
    sid = lax.axis_index("s")
    wid = sid * _NC + cid
    pltpu.sync_copy(col_hbm.at[wid], col_v)
    pltpu.sync_copy(ew_hbm.at[wid], ew_v)

    def zero_body(i, carry):
        deg_v[pl.ds(i * 16, 16)] = jnp.zeros((16,), jnp.float32)
        return carry

    lax.fori_loop(0, _N // 16, zero_body, 0)

    def acc_body(i, carry):
        c = col_v[pl.ds(i * 16, 16)]
        w = ew_v[pl.ds(i * 16, 16)]
        plsc.addupdate_scatter(deg_v, [c], w)
        return carry

    lax.fori_loop(0, _EP // 16, acc_body, 0)
    pltpu.sync_copy(deg_v, out_hbm.at[wid])


@functools.partial(
    pl.kernel,
    out_type=jax.ShapeDtypeStruct((_NC, _N, _D), jnp.float32),
    mesh=_sc_mesh(),
    compiler_params=_SC_PARAMS,
    scratch_types=[
        pltpu.VMEM((2, _C), jnp.int32),
        pltpu.VMEM((2, _C), jnp.int32),
        pltpu.VMEM((2, _C), jnp.int32),
        pltpu.VMEM((_C, _D), jnp.float32),
        pltpu.VMEM((_C, _D), jnp.float32),
        pltpu.VMEM((_C, _D), jnp.float32),
        pltpu.VMEM((_C // 8, _D), jnp.float32),
        pltpu.VMEM((_C // 8, _D), jnp.float32),
        pltpu.VMEM((_C // 8, _D), jnp.float32),
        pltpu.VMEM_SHARED((_N, _D), jnp.float32),
        pltpu.SemaphoreType.DMA, pltpu.SemaphoreType.DMA,
        pltpu.SemaphoreType.DMA, pltpu.SemaphoreType.DMA,
        pltpu.SemaphoreType.DMA, pltpu.SemaphoreType.DMA,
        pltpu.SemaphoreType.DMA, pltpu.SemaphoreType.DMA,
        pltpu.SemaphoreType.DMA, pltpu.SemaphoreType.DMA,
        pltpu.SemaphoreType.DMA, pltpu.SemaphoreType.DMA,
    ],
)
def _msg_kernel(ei_hbm, ewb_hbm, xs_hbm, out_hbm,
                ib0, ib1, ib2, gb0, gb1, gb2, eb0, eb1, eb2, acc_s,
                is0, is1, is2, gs0, gs1, gs2, es0, es1, es2,
                ss0, ss1, ss2):
    cid = lax.axis_index("c")
    sid = lax.axis_index("s")
    wid = sid * _NC + cid
    ib = (ib0, ib1, ib2)
    gb = (gb0, gb1, gb2)
    eb = (eb0, eb1, eb2)
    isem = (is0, is1, is2)
    gsem = (gs0, gs1, gs2)
    esem = (es0, es1, es2)
    ssem = (ss0, ss1, ss2)

    def zb(i, carry):
        r = i // 8
        l = i % 8
        gb0[r, pl.ds(l * 16, 16)] = jnp.zeros((16,), jnp.float32)
        return carry

    lax.fori_loop(0, _C * (_D // 16), zb, 0)

    def zs(i, carry):
        pltpu.sync_copy(gb0, acc_s.at[pl.ds(sid * _RPT + i * _C, _C)])
        return carry

    lax.fori_loop(0, _RPT // _C, zs, 0)
    rem = _RPT % _C
    if rem:
        pltpu.sync_copy(gb0.at[pl.ds(0, rem)],
                        acc_s.at[pl.ds(sid * _RPT + (_RPT // _C) * _C, rem)])
    plsc.subcore_barrier()

    def start_idx(j, b):
        base = wid * _EP + j * _C
        pltpu.async_copy(ei_hbm.at[0, pl.ds(base, _C)], ib[b].at[0], isem[b])
        pltpu.async_copy(ei_hbm.at[1, pl.ds(base, _C)], ib[b].at[1], isem[b])

    def wait_idx(b):
        pltpu.make_async_copy(ei_hbm.at[0, pl.ds(0, _C)], ib[b].at[0],
                              isem[b]).wait()
        pltpu.make_async_copy(ei_hbm.at[1, pl.ds(0, _C)], ib[b].at[1],
                              isem[b]).wait()

    def start_gather(j, b):
        pltpu.async_copy(xs_hbm.at[ib[b].at[0]], gb[b], gsem[b])
        erow = wid * (_EP // 8) + j * (_C // 8)
        pltpu.async_copy(ewb_hbm.at[pl.ds(erow, _C // 8)], eb[b], esem[b])

    def wait_gather(b):
        pltpu.make_async_copy(xs_hbm.at[ib[b].at[0]], gb[b], gsem[b]).wait()
        pltpu.make_async_copy(ewb_hbm.at[pl.ds(0, _C // 8)], eb[b],
                              esem[b]).wait()

    def scale(b):
        def srow(k, carry):
            bw = eb[b][k // 8, pl.ds((k % 8) * 16, 16)]
            for m in range(_D // 16):
                sl = pl.ds(m * 16, 16)
                gb[b][k, sl] = gb[b][k, sl] * bw
            return carry

        lax.fori_loop(0, _C, srow, 0)

    def start_scatter(j, b):
        pltpu.async_copy(gb[b], acc_s.at[ib[b].at[1]], ssem[b], add=True)

    def wait_scatter(b):
        pltpu.make_async_copy(gb[b], acc_s.at[ib[b].at[1]], ssem[b]).wait()

    start_idx(0, 0)
    wait_idx(0)
    start_idx(1, 1)
    start_gather(0, 0)

    def step(j, x, p, n, first, tail):
        if not tail:
            @pl.when(j < _NCHUNK - 1)
            def _():
                wait_idx(n)
                start_gather(j + 1, n)
        wait_gather(x)
        scale(x)
        if not first:
            wait_scatter(p)
        if not tail:
            @pl.when(j < _NCHUNK - 2)
            def _():
                start_idx(j + 2, p)
        start_scatter(j, x)

    def triple(j3, carry):
        j = j3 * 3
        step(j, 0, 2, 1, False, False)
        step(j + 1, 1, 0, 2, False, False)
        step(j + 2, 2, 1, 0, False, False)
        return carry

    step(0, 0, 2, 1, True, False)
    step(1, 1, 0, 2, False, False)
    step(2, 2, 1, 0, False, False)
    lax.fori_loop(1, _NCHUNK // 3, triple, 0)
    step(_NCHUNK - 2, (_NCHUNK - 2) % 3, (_NCHUNK - 3) % 3,
         (_NCHUNK - 1) % 3, False, False)
    step(_NCHUNK - 1, (_NCHUNK - 1) % 3, (_NCHUNK - 2) % 3,
         _NCHUNK % 3, False, True)
    wait_scatter((_NCHUNK - 1) % 3)
    plsc.subcore_barrier()

    def co(i, carry):
        base = sid * _RPT + i * _ZR
        pltpu.sync_copy(acc_s.at[pl.ds(base, _ZR)],
                        out_hbm.at[cid, pl.ds(base, _ZR)])
        return carry

    lax.fori_loop(0, _NZ, co, 0)


_EPACK = 8
_ER = _E // _EPACK
_EB = 5000


def _edge_mlp_body(ea_ref, w1_ref, b1_ref, w2_ref, b2_ref, w2c_ref, b2c_ref,
                   out_ref, outc_ref):
    h = jnp.dot(ea_ref[...], w1_ref[...], preferred_element_type=jnp.float32)
    h = jax.nn.relu(h + b1_ref[...])
    ew = jnp.dot(h, w2_ref[...], preferred_element_type=jnp.float32) + b2_ref[...]
    out_ref[...] = ew
    ewc = jnp.dot(h, w2c_ref[...], preferred_element_type=jnp.float32) + b2c_ref[...]
    outc_ref[...] = ewc


def _edge_mlp(edge_attr, w1, b1, w2, b2):
    ea2 = edge_attr.reshape(_ER, _EPACK * 16)
    eye = jnp.eye(_EPACK, dtype=jnp.float32)
    w1b = jnp.einsum("pq,io->piqo", eye, w1).reshape(_EPACK * 16, _EPACK * 16)
    b1t = jnp.tile(b1, _EPACK).reshape(1, _EPACK * 16)
    ones16 = jnp.ones((16,), jnp.float32)
    w2b = jnp.einsum("pq,i,m->piqm", eye, w2[:, 0],
                     ones16).reshape(_EPACK * 16, _EPACK * 16)
    b2t = jnp.broadcast_to(b2, (1, _EPACK * 16))
    w2c = jnp.einsum("pq,io->piqo", eye, w2).reshape(_EPACK * 16, _EPACK)
    b2c = jnp.tile(b2, _EPACK).reshape(1, _EPACK)
    grid = _ER // _EB
    return pl.pallas_call(
        _edge_mlp_body,
        grid=(grid,),
        in_specs=[
            pl.BlockSpec((_EB, _EPACK * 16), lambda i: (i, 0)),
            pl.BlockSpec((_EPACK * 16, _EPACK * 16), lambda i: (0, 0)),
            pl.BlockSpec((1, _EPACK * 16), lambda i: (0, 0)),
            pl.BlockSpec((_EPACK * 16, _EPACK * 16), lambda i: (0, 0)),
            pl.BlockSpec((1, _EPACK * 16), lambda i: (0, 0)),
            pl.BlockSpec((_EPACK * 16, _EPACK), lambda i: (0, 0)),
            pl.BlockSpec((1, _EPACK), lambda i: (0, 0)),
        ],
        out_specs=[
            pl.BlockSpec((_EB, _EPACK * 16), lambda i: (i, 0)),
            pl.BlockSpec((_EB, _EPACK), lambda i: (i, 0)),
        ],
        out_shape=[
            jax.ShapeDtypeStruct((_ER, _EPACK * 16), jnp.float32),
            jax.ShapeDtypeStruct((_ER, _EPACK), jnp.float32),
        ],
    )(ea2, w1b, b1t, w2b, b2t, w2c, b2c)


def _prep_body(degp_ref, x_ref, w_ref, dinv_ref, xs_ref):
    deg = jnp.sum(degp_ref[...], axis=0) + 1.0
    dinv = jnp.where(deg > 0,
                     lax.rsqrt(jnp.maximum(deg, 1e-12)),
                     0.0)
    dinv_ref[...] = dinv.reshape(1, _N)
    xw = jnp.dot(x_ref[...], w_ref[...], preferred_element_type=jnp.float32)
    xs_ref[...] = dinv.reshape(_N, 1) * xw


def _prep(deg_partials, x, conv1_w):
    return pl.pallas_call(
        _prep_body,
        out_shape=(
            jax.ShapeDtypeStruct((1, _N), jnp.float32),
            jax.ShapeDtypeStruct((_N, _D), jnp.float32),
        ),
    )(deg_partials, x, conv1_w)


def _mid_body(acc_ref, xs_ref, dinv_ref, b_ref, w_ref, xs2_ref):
    dinv = dinv_ref[...].reshape(_N, 1)
    tot = acc_ref[0] + acc_ref[1] + xs_ref[...]
    x1 = jax.nn.relu(dinv * tot + b_ref[...])
    xw2 = jnp.dot(x1, w_ref[...], preferred_element_type=jnp.float32)
    xs2_ref[...] = dinv * xw2


def _mid(acc1, xs1, dinv, conv1_b, conv2_w):
    return pl.pallas_call(
        _mid_body,
        out_shape=jax.ShapeDtypeStruct((_N, _D), jnp.float32),
    )(acc1, xs1, dinv, conv1_b.reshape(1, _D), conv2_w)


def _tail_body(acc_ref, xs_ref, dinv_ref, b_ref, bm_ref,
               l1w_ref, l1b_ref, lmw_ref, lmb_ref, mu_ref):
    dinv = dinv_ref[...].reshape(_N, 1)
    tot = acc_ref[0] + acc_ref[1] + xs_ref[...]
    x2 = jax.nn.relu(dinv * tot + b_ref[...])
    seg = lax.broadcasted_iota(jnp.int32, (_G, _N), 0)
    onehot = (seg == bm_ref[...]).astype(jnp.float32)
    sums = jnp.dot(onehot, x2, preferred_element_type=jnp.float32)
    cnt = jnp.sum(onehot, axis=1, keepdims=True)
    pooled = sums / jnp.maximum(cnt, 1.0)
    emb = jax.nn.relu(jnp.dot(pooled, l1w_ref[...], preferred_element_type=jnp.float32) + l1b_ref[...])
    mu_ref[...] = jnp.dot(emb, lmw_ref[...], preferred_element_type=jnp.float32) + lmb_ref[...]


def _tail(acc2, xs2, dinv, conv2_b, batch_mask, l1w, l1b, lmw, lmb):
    return pl.pallas_call(
        _tail_body,
        out_shape=jax.ShapeDtypeStruct((_G, 64), jnp.float32),
    )(acc2, xs2, dinv, conv2_b.reshape(1, _D), batch_mask.reshape(1, _N),
      l1w, l1b.reshape(1, _D), lmw, lmb.reshape(1, 64))


@jax.jit
def _run(x, edge_index, edge_attr, batch_mask, nn_W1, nn_b1, nn_W2, nn_b2,
         conv1_W, conv1_b, conv2_W, conv2_b, lin1_W, lin1_b,
         lin_mu_W, lin_mu_b):
    ewb, ewc = _edge_mlp(edge_attr, nn_W1, nn_b1, nn_W2, nn_b2)
    ew = ewc.reshape(_NW, _EP)
    deg_partials = _deg_kernel(edge_index[1].reshape(_NW, _EP), ew)
    dinv, xs1 = _prep(deg_partials, x, conv1_W)
    acc1 = _msg_kernel(edge_index, ewb, xs1)
    xs2 = _mid(acc1, xs1, dinv, conv1_b, conv2_W)
    acc2 = _msg_kernel(edge_index, ewb, xs2)
    return _tail(acc2, xs2, dinv, conv2_b, batch_mask,
                 lin1_W, lin1_b, lin_mu_W, lin_mu_b)


def kernel(x, edge_index, edge_attr, batch_mask, nn_W1, nn_b1, nn_W2, nn_b2,
           conv1_W, conv1_b, conv2_W, conv2_b, lin1_W, lin1_b,
           lin_mu_W, lin_mu_b):
    return _run(x, edge_index, edge_attr, batch_mask, nn_W1, nn_b1, nn_W2,
                nn_b2, conv1_W, conv1_b, conv2_W, conv2_b, lin1_W, lin1_b,
                lin_mu_W, lin_mu_b)


# speedup vs baseline: 18.3228x; 1.0186x over previous
import functools

import jax
import jax.numpy as jnp
from jax import lax
from jax.experimental import pallas as pl
from jax.experimental.pallas import tpu as pltpu
from jax.experimental.pallas import tpu_sc as plsc

_N = 10000
_E = 320000
_D = 128
_G = 64
_NC = 2
_NS = 16
_NW = _NC * _NS
_EP = _E // _NW
_C = 80
_NCHUNK = _EP // _C
_RPT = _N // _NS
_ZR = 125
_NZ = _RPT // _ZR


def _sc_mesh():
    return plsc.VectorSubcoreMesh(core_axis_name="c", subcore_axis_name="s")


_SC_PARAMS = pltpu.CompilerParams(
    needs_layout_passes=False, use_tc_tiling_on_sc=False
)


_DP = 5
_DR = _EP // 8 // _DP
_DG = _DR * 8 // 16


@functools.partial(
    pl.kernel,
    out_type=jax.ShapeDtypeStruct((_NW, _N), jnp.float32),
    mesh=_sc_mesh(),
    compiler_params=_SC_PARAMS,
    scratch_types=[
        pltpu.VMEM((_N,), jnp.float32),
        pltpu.VMEM((_EP,), jnp.int32),
        pltpu.VMEM((_DR, _D), jnp.float32),
    ],
)
def _deg_kernel(ei_hbm, ewb_hbm, out_hbm, deg_v, col_v, ew_v):
    cid = lax.axis_index("c")
    sid = lax.axis_index("s")
    wid = sid * _NC + cid
    pltpu.sync_copy(ei_hbm.at[1, pl.ds(wid * _EP, _EP)], col_v)

    def zero_body(i, carry):
        deg_v[pl.ds(i * 16, 16)] = jnp.zeros((16,), jnp.float32)
        return carry

    lax.fori_loop(0, _N // 16, zero_body, 0)

    lane = lax.broadcasted_iota(jnp.int32, (16,), 0)

    def dpass(p, carry):
        pltpu.sync_copy(ewb_hbm.at[pl.ds(wid * (_EP // 8) + p * _DR, _DR)],
                        ew_v)

        def acc_body(g, c2):
            e = g * 16 + lane
            ridx = e // 8
            lidx = (e % 8) * 16
            w = plsc.load_gather(ew_v, [ridx, lidx])
            c = col_v[pl.ds(p * (_DG * 16) + g * 16, 16)]
            plsc.addupdate_scatter(deg_v, [c], w)
            return c2

        lax.fori_loop(0, _DG, acc_body, 0)
        return carry

    lax.fori_loop(0, _DP, dpass, 0)
    pltpu.sync_copy(deg_v, out_hbm.at[wid])


@functools.partial(
    pl.kernel,
    out_type=jax.ShapeDtypeStruct((_NC, _N, _D), jnp.float32),
    mesh=_sc_mesh(),
    compiler_params=_SC_PARAMS,
    scratch_types=[
        pltpu.VMEM((2, _C), jnp.int32),
        pltpu.VMEM((2, _C), jnp.int32),
        pltpu.VMEM((2, _C), jnp.int32),
        pltpu.VMEM((_C, _D), jnp.float32),
        pltpu.VMEM((_C, _D), jnp.float32),
        pltpu.VMEM((_C, _D), jnp.float32),
        pltpu.VMEM((_C // 8, _D), jnp.float32),
        pltpu.VMEM((_C // 8, _D), jnp.float32),
        pltpu.VMEM((_C // 8, _D), jnp.float32),
        pltpu.VMEM_SHARED((_N, _D), jnp.float32),
        pltpu.SemaphoreType.DMA, pltpu.SemaphoreType.DMA,
        pltpu.SemaphoreType.DMA, pltpu.SemaphoreType.DMA,
        pltpu.SemaphoreType.DMA, pltpu.SemaphoreType.DMA,
        pltpu.SemaphoreType.DMA, pltpu.SemaphoreType.DMA,
        pltpu.SemaphoreType.DMA, pltpu.SemaphoreType.DMA,
        pltpu.SemaphoreType.DMA, pltpu.SemaphoreType.DMA,
    ],
)
def _msg_kernel(ei_hbm, ewb_hbm, xs_hbm, out_hbm,
                ib0, ib1, ib2, gb0, gb1, gb2, eb0, eb1, eb2, acc_s,
                is0, is1, is2, gs0, gs1, gs2, es0, es1, es2,
                ss0, ss1, ss2):
    cid = lax.axis_index("c")
    sid = lax.axis_index("s")
    wid = sid * _NC + cid
    ib = (ib0, ib1, ib2)
    gb = (gb0, gb1, gb2)
    eb = (eb0, eb1, eb2)
    isem = (is0, is1, is2)
    gsem = (gs0, gs1, gs2)
    esem = (es0, es1, es2)
    ssem = (ss0, ss1, ss2)

    def zb(i, carry):
        r = i // 8
        l = i % 8
        gb0[r, pl.ds(l * 16, 16)] = jnp.zeros((16,), jnp.float32)
        return carry

    lax.fori_loop(0, _C * (_D // 16), zb, 0)

    def zs(i, carry):
        pltpu.sync_copy(gb0, acc_s.at[pl.ds(sid * _RPT + i * _C, _C)])
        return carry

    lax.fori_loop(0, _RPT // _C, zs, 0)
    rem = _RPT % _C
    if rem:
        pltpu.sync_copy(gb0.at[pl.ds(0, rem)],
                        acc_s.at[pl.ds(sid * _RPT + (_RPT // _C) * _C, rem)])
    plsc.subcore_barrier()

    def start_idx(j, b):
        base = wid * _EP + j * _C
        pltpu.async_copy(ei_hbm.at[0, pl.ds(base, _C)], ib[b].at[0], isem[b])
        pltpu.async_copy(ei_hbm.at[1, pl.ds(base, _C)], ib[b].at[1], isem[b])

    def wait_idx(b):
        pltpu.make_async_copy(ei_hbm.at[0, pl.ds(0, _C)], ib[b].at[0],
                              isem[b]).wait()
        pltpu.make_async_copy(ei_hbm.at[1, pl.ds(0, _C)], ib[b].at[1],
                              isem[b]).wait()

    def start_gather(j, b):
        pltpu.async_copy(xs_hbm.at[ib[b].at[0]], gb[b], gsem[b])
        erow = wid * (_EP // 8) + j * (_C // 8)
        pltpu.async_copy(ewb_hbm.at[pl.ds(erow, _C // 8)], eb[b], esem[b])

    def wait_gather(b):
        pltpu.make_async_copy(xs_hbm.at[ib[b].at[0]], gb[b], gsem[b]).wait()
        pltpu.make_async_copy(ewb_hbm.at[pl.ds(0, _C // 8)], eb[b],
                              esem[b]).wait()

    def scale(b):
        def srow(k, carry):
            bw = eb[b][k // 8, pl.ds((k % 8) * 16, 16)]
            for m in range(_D // 16):
                sl = pl.ds(m * 16, 16)
                gb[b][k, sl] = gb[b][k, sl] * bw
            return carry

        lax.fori_loop(0, _C, srow, 0)

    def start_scatter(j, b):
        pltpu.async_copy(gb[b], acc_s.at[ib[b].at[1]], ssem[b], add=True)

    def wait_scatter(b):
        pltpu.make_async_copy(gb[b], acc_s.at[ib[b].at[1]], ssem[b]).wait()

    start_idx(0, 0)
    wait_idx(0)
    start_idx(1, 1)
    start_gather(0, 0)

    def step(j, x, p, n, first, tail):
        if not tail:
            @pl.when(j < _NCHUNK - 1)
            def _():
                wait_idx(n)
                start_gather(j + 1, n)
        wait_gather(x)
        scale(x)
        if not first:
            wait_scatter(p)
        if not tail:
            @pl.when(j < _NCHUNK - 2)
            def _():
                start_idx(j + 2, p)
        start_scatter(j, x)

    def triple(j3, carry):
        j = j3 * 3
        step(j, 0, 2, 1, False, False)
        step(j + 1, 1, 0, 2, False, False)
        step(j + 2, 2, 1, 0, False, False)
        return carry

    step(0, 0, 2, 1, True, False)
    step(1, 1, 0, 2, False, False)
    step(2, 2, 1, 0, False, False)
    lax.fori_loop(1, _NCHUNK // 3, triple, 0)
    step(_NCHUNK - 2, (_NCHUNK - 2) % 3, (_NCHUNK - 3) % 3,
         (_NCHUNK - 1) % 3, False, False)
    step(_NCHUNK - 1, (_NCHUNK - 1) % 3, (_NCHUNK - 2) % 3,
         _NCHUNK % 3, False, True)
    wait_scatter((_NCHUNK - 1) % 3)
    plsc.subcore_barrier()

    def co(i, carry):
        base = sid * _RPT + i * _ZR
        pltpu.sync_copy(acc_s.at[pl.ds(base, _ZR)],
                        out_hbm.at[cid, pl.ds(base, _ZR)])
        return carry

    lax.fori_loop(0, _NZ, co, 0)


_EPACK = 8
_ER = _E // _EPACK
_EB = 5000


def _edge_mlp_body(ea_ref, w1_ref, b1_ref, w2_ref, b2_ref, out_ref):
    h = jnp.dot(ea_ref[...], w1_ref[...], preferred_element_type=jnp.float32)
    h = jax.nn.relu(h + b1_ref[...])
    ew = jnp.dot(h, w2_ref[...], preferred_element_type=jnp.float32) + b2_ref[...]
    out_ref[...] = ew


def _edge_mlp(edge_attr, w1, b1, w2, b2):
    ea2 = edge_attr.reshape(_ER, _EPACK * 16)
    eye = jnp.eye(_EPACK, dtype=jnp.float32)
    w1b = jnp.einsum("pq,io->piqo", eye, w1).reshape(_EPACK * 16, _EPACK * 16)
    b1t = jnp.tile(b1, _EPACK).reshape(1, _EPACK * 16)
    ones16 = jnp.ones((16,), jnp.float32)
    w2b = jnp.einsum("pq,i,m->piqm", eye, w2[:, 0],
                     ones16).reshape(_EPACK * 16, _EPACK * 16)
    b2t = jnp.broadcast_to(b2, (1, _EPACK * 16))
    grid = _ER // _EB
    return pl.pallas_call(
        _edge_mlp_body,
        grid=(grid,),
        in_specs=[
            pl.BlockSpec((_EB, _EPACK * 16), lambda i: (i, 0)),
            pl.BlockSpec((_EPACK * 16, _EPACK * 16), lambda i: (0, 0)),
            pl.BlockSpec((1, _EPACK * 16), lambda i: (0, 0)),
            pl.BlockSpec((_EPACK * 16, _EPACK * 16), lambda i: (0, 0)),
            pl.BlockSpec((1, _EPACK * 16), lambda i: (0, 0)),
        ],
        out_specs=pl.BlockSpec((_EB, _EPACK * 16), lambda i: (i, 0)),
        out_shape=jax.ShapeDtypeStruct((_ER, _EPACK * 16), jnp.float32),
    )(ea2, w1b, b1t, w2b, b2t)


def _prep_body(degp_ref, x_ref, w_ref, dinv_ref, xs_ref):
    deg = jnp.sum(degp_ref[...], axis=0) + 1.0
    dinv = jnp.where(deg > 0,
                     lax.rsqrt(jnp.maximum(deg, 1e-12)),
                     0.0)
    dinv_ref[...] = dinv.reshape(1, _N)
    xw = jnp.dot(x_ref[...], w_ref[...], preferred_element_type=jnp.float32)
    xs_ref[...] = dinv.reshape(_N, 1) * xw


def _prep(deg_partials, x, conv1_w):
    return pl.pallas_call(
        _prep_body,
        out_shape=(
            jax.ShapeDtypeStruct((1, _N), jnp.float32),
            jax.ShapeDtypeStruct((_N, _D), jnp.float32),
        ),
    )(deg_partials, x, conv1_w)


def _mid_body(acc_ref, xs_ref, dinv_ref, b_ref, w_ref, xs2_ref):
    dinv = dinv_ref[...].reshape(_N, 1)
    tot = acc_ref[0] + acc_ref[1] + xs_ref[...]
    x1 = jax.nn.relu(dinv * tot + b_ref[...])
    xw2 = jnp.dot(x1, w_ref[...], preferred_element_type=jnp.float32)
    xs2_ref[...] = dinv * xw2


def _mid(acc1, xs1, dinv, conv1_b, conv2_w):
    return pl.pallas_call(
        _mid_body,
        out_shape=jax.ShapeDtypeStruct((_N, _D), jnp.float32),
    )(acc1, xs1, dinv, conv1_b.reshape(1, _D), conv2_w)


def _tail_body(acc_ref, xs_ref, dinv_ref, b_ref, bm_ref,
               l1w_ref, l1b_ref, lmw_ref, lmb_ref, mu_ref):
    dinv = dinv_ref[...].reshape(_N, 1)
    tot = acc_ref[0] + acc_ref[1] + xs_ref[...]
    x2 = jax.nn.relu(dinv * tot + b_ref[...])
    seg = lax.broadcasted_iota(jnp.int32, (_G, _N), 0)
    onehot = (seg == bm_ref[...]).astype(jnp.float32)
    sums = jnp.dot(onehot, x2, preferred_element_type=jnp.float32)
    cnt = jnp.sum(onehot, axis=1, keepdims=True)
    pooled = sums / jnp.maximum(cnt, 1.0)
    emb = jax.nn.relu(jnp.dot(pooled, l1w_ref[...], preferred_element_type=jnp.float32) + l1b_ref[...])
    mu_ref[...] = jnp.dot(emb, lmw_ref[...], preferred_element_type=jnp.float32) + lmb_ref[...]


def _tail(acc2, xs2, dinv, conv2_b, batch_mask, l1w, l1b, lmw, lmb):
    return pl.pallas_call(
        _tail_body,
        out_shape=jax.ShapeDtypeStruct((_G, 64), jnp.float32),
    )(acc2, xs2, dinv, conv2_b.reshape(1, _D), batch_mask.reshape(1, _N),
      l1w, l1b.reshape(1, _D), lmw, lmb.reshape(1, 64))


@jax.jit
def _run(x, edge_index, edge_attr, batch_mask, nn_W1, nn_b1, nn_W2, nn_b2,
         conv1_W, conv1_b, conv2_W, conv2_b, lin1_W, lin1_b,
         lin_mu_W, lin_mu_b):
    ewb = _edge_mlp(edge_attr, nn_W1, nn_b1, nn_W2, nn_b2)
    deg_partials = _deg_kernel(edge_index, ewb)
    dinv, xs1 = _prep(deg_partials, x, conv1_W)
    acc1 = _msg_kernel(edge_index, ewb, xs1)
    xs2 = _mid(acc1, xs1, dinv, conv1_b, conv2_W)
    acc2 = _msg_kernel(edge_index, ewb, xs2)
    return _tail(acc2, xs2, dinv, conv2_b, batch_mask,
                 lin1_W, lin1_b, lin_mu_W, lin_mu_b)


def kernel(x, edge_index, edge_attr, batch_mask, nn_W1, nn_b1, nn_W2, nn_b2,
           conv1_W, conv1_b, conv2_W, conv2_b, lin1_W, lin1_b,
           lin_mu_W, lin_mu_b):
    return _run(x, edge_index, edge_attr, batch_mask, nn_W1, nn_b1, nn_W2,
                nn_b2, conv1_W, conv1_b, conv2_W, conv2_b, lin1_W, lin1_b,
                lin_mu_W, lin_mu_b)
